# baseline (speedup 1.0000x reference)
import functools
import jax
import jax.numpy as jnp
from jax import lax
from jax.experimental import pallas as pl
from jax.experimental.pallas import tpu as pltpu
from jax.experimental.pallas import tpu_sc as plsc

N_NODES = 100000
NPAD = 100352
N_SC = 2
N_TILE = 16
DH = 16

EDGE_ALIGN = 32 * 2048

DEG_SLAB = 2048
AGG_CHUNK = 128
AGG_NCHUNK = 2
AGG_SLAB = AGG_CHUNK * AGG_NCHUNK
DEPTH = 4


def _sc_mesh():
    return plsc.VectorSubcoreMesh(
        core_axis_name="c", subcore_axis_name="s",
        num_cores=N_SC, num_subcores=N_TILE)


def _deg_body(dst_hbm, parts_hbm, dbuf, acc):
    ep = dst_hbm.shape[0]
    wid = lax.axis_index("s") * N_SC + lax.axis_index("c")
    ew = ep // (N_SC * N_TILE)
    base0 = wid * ew
    ones = jnp.full((16,), 1.0, jnp.float32)
    zeros = jnp.zeros((16,), jnp.float32)

    def zbody(i, _):
        acc[pl.ds(i * 16, 16)] = zeros
        return 0
    lax.fori_loop(0, NPAD // 16, zbody, 0)

    def slab(i, _):
        pltpu.sync_copy(dst_hbm.at[pl.ds(base0 + i * DEG_SLAB, DEG_SLAB)], dbuf)

        def inner(j, _):
            idx = dbuf[pl.ds(j * 16, 16)]
            plsc.addupdate_scatter(acc, [idx], ones)
            return 0
        lax.fori_loop(0, DEG_SLAB // 16, inner, 0)
        return 0
    lax.fori_loop(0, ew // DEG_SLAB, slab, 0)

    pltpu.sync_copy(acc, parts_hbm.at[pl.ds(wid * NPAD, NPAD)])


def _deg_call(dst_p):
    return pl.kernel(
        _deg_body,
        out_type=jax.ShapeDtypeStruct((N_SC * N_TILE * NPAD,), jnp.float32),
        mesh=_sc_mesh(),
        compiler_params=pltpu.CompilerParams(needs_layout_passes=False, use_tc_tiling_on_sc=False),
        scratch_types=[
            pltpu.VMEM((DEG_SLAB,), jnp.int32),
            pltpu.VMEM((NPAD,), jnp.float32),
        ],
    )(dst_p)


def _agg_body(g_hbm, src_hbm, dst_hbm, out_hbm,
              sbuf, dbuf, rows, acc_sh, semi, semg, sems):
    nrows_idx = src_hbm.shape[0]
    c = lax.axis_index("c")
    s = lax.axis_index("s")
    gtab = g_hbm.at[c]
    otab = out_hbm.at[c]

    zeros = jnp.zeros((16,), jnp.float32)

    def zfill(i, _):
        rows[0, i] = zeros
        return 0
    lax.fori_loop(0, AGG_SLAB, zfill, 0)

    stripe = NPAD // N_TILE
    r0 = s * stripe
    nz_full, nz_rem = divmod(stripe, AGG_SLAB)
    for k in range(nz_full):
        pltpu.sync_copy(rows.at[0],
                        acc_sh.at[pl.ds(r0 + k * AGG_SLAB, AGG_SLAB), :])
    if nz_rem:
        pltpu.sync_copy(rows.at[0].at[pl.ds(0, nz_rem), :],
                        acc_sh.at[pl.ds(r0 + nz_full * AGG_SLAB, nz_rem), :])
    plsc.subcore_barrier()

    rows_per_tile = nrows_idx // N_TILE
    nslab = rows_per_tile // AGG_NCHUNK
    irow0 = s * rows_per_tile

    def issue_idx(slot, i):
        ib = irow0 + i * AGG_NCHUNK
        pltpu.async_copy(src_hbm.at[pl.ds(ib, AGG_NCHUNK), :], sbuf.at[slot], semi)
        pltpu.async_copy(dst_hbm.at[pl.ds(ib, AGG_NCHUNK), :], dbuf.at[slot], semi)

    def wait_idx(slot):
        pltpu.make_async_copy(
            src_hbm.at[pl.ds(0, AGG_NCHUNK), :], sbuf.at[slot], semi).wait()
        pltpu.make_async_copy(
            dst_hbm.at[pl.ds(0, AGG_NCHUNK), :], dbuf.at[slot], semi).wait()

    def fire_gathers(slot):
        for k in range(AGG_NCHUNK):
            pltpu.async_copy(
                gtab.at[sbuf.at[slot].at[k]],
                rows.at[slot].at[pl.ds(k * AGG_CHUNK, AGG_CHUNK), :], semg)

    def drain_gathers(slot):
        for k in range(AGG_NCHUNK):
            pltpu.make_async_copy(
                gtab.at[sbuf.at[slot].at[k]],
                rows.at[slot].at[pl.ds(k * AGG_CHUNK, AGG_CHUNK), :], semg).wait()

    def fire_scatters(slot):
        for k in range(AGG_NCHUNK):
            pltpu.async_copy(
                rows.at[slot].at[pl.ds(k * AGG_CHUNK, AGG_CHUNK), :],
                acc_sh.at[dbuf.at[slot].at[k]], sems, add=True)

    def drain_scatter(slot):
        for k in range(AGG_NCHUNK):
            pltpu.make_async_copy(
                rows.at[slot].at[pl.ds(k * AGG_CHUNK, AGG_CHUNK), :],
                acc_sh.at[dbuf.at[slot].at[k]], sems).wait()

    issue_idx(0, 0)
    issue_idx(1, 1)
    wait_idx(0)
    fire_gathers(0)

    def body(i4, _):
        for u in range(4):
            i = i4 * 4 + u

            @pl.when(i >= 2)
            def _():
                drain_scatter((u - 2) % 4)

            @pl.when(i + 2 < nslab)
            def _():
                issue_idx((u + 2) % 4, i + 2)

            @pl.when(i + 1 < nslab)
            def _():
                wait_idx((u + 1) % 4)
                fire_gathers((u + 1) % 4)

            drain_gathers(u)
            fire_scatters(u)
        return 0
    lax.fori_loop(0, nslab // 4, body, 0)
    drain_scatter(2)
    drain_scatter(3)
    plsc.subcore_barrier()

    pltpu.sync_copy(acc_sh.at[pl.ds(r0, stripe), :], otab.at[pl.ds(r0, stripe), :])


def _agg_call(g, src2d, dst2d):
    return pl.kernel(
        _agg_body,
        out_type=jax.ShapeDtypeStruct((N_SC, NPAD, DH), jnp.float32),
        mesh=_sc_mesh(),
        compiler_params=pltpu.CompilerParams(needs_layout_passes=False, use_tc_tiling_on_sc=False),
        scratch_types=[
            pltpu.VMEM((DEPTH, AGG_NCHUNK, AGG_CHUNK), jnp.int32),
            pltpu.VMEM((DEPTH, AGG_NCHUNK, AGG_CHUNK), jnp.int32),
            pltpu.VMEM((DEPTH, AGG_SLAB, DH), jnp.float32),
            pltpu.VMEM_SHARED((NPAD, DH), jnp.float32),
            pltpu.SemaphoreType.DMA,
            pltpu.SemaphoreType.DMA,
            pltpu.SemaphoreType.DMA,
        ],
    )(g, src2d, dst2d)


_R = 2048
_G = (N_NODES + _R - 1) // _R


_PK = NPAD // 8
_RP = _R // 8
_DK = NPAD // 128
_GD = _DK // 16


def _dinv_body(parts_ref, dinv_ref):
    deg = jnp.sum(parts_ref[...], axis=0) + 1.0
    dinv_ref[...] = lax.rsqrt(deg)


def _dinv_call(parts3):
    return pl.pallas_call(
        _dinv_body,
        grid=(_GD,),
        in_specs=[pl.BlockSpec((N_SC * N_TILE, 16, 128), lambda i: (0, i, 0))],
        out_specs=pl.BlockSpec((16, 128), lambda i: (i, 0)),
        out_shape=jax.ShapeDtypeStruct((_DK, 128), jnp.float32),
    )(parts3)


def _prep1_body(xp_ref, d16_ref, wlo_ref, whi_ref, g_ref):
    xb = xp_ref[...]
    d16 = d16_ref[...]
    g_ref[0] = jnp.dot(xb, wlo_ref[...], preferred_element_type=jnp.float32) * d16
    g_ref[1] = jnp.dot(xb, whi_ref[...], preferred_element_type=jnp.float32) * d16


def _prep1_call(xp, d16, w1lo, w1hi):
    return pl.pallas_call(
        _prep1_body,
        grid=(_G,),
        in_specs=[
            pl.BlockSpec((_RP, 72), lambda i: (i, 0)),
            pl.BlockSpec((_RP, 128), lambda i: (i, 0)),
            pl.BlockSpec((72, 128), lambda i: (0, 0)),
            pl.BlockSpec((72, 128), lambda i: (0, 0)),
        ],
        out_specs=pl.BlockSpec((N_SC, _RP, 128), lambda i: (0, i, 0)),
        out_shape=jax.ShapeDtypeStruct((N_SC, _PK, 128), jnp.float32),
    )(xp, d16, w1lo, w1hi)


def _mid_body(acc_ref, g_ref, d16_ref, blo_ref, bhi_ref, wlo_ref, whi_ref,
              g2_ref):
    d16 = d16_ref[...]
    t_lo = jnp.maximum((acc_ref[0] + g_ref[0]) * d16 + blo_ref[...], 0.0)
    t_hi = jnp.maximum((acc_ref[1] + g_ref[1]) * d16 + bhi_ref[...], 0.0)
    hcat = jnp.concatenate([t_lo, t_hi], axis=1)
    g2_ref[0] = jnp.dot(
        hcat, wlo_ref[...], preferred_element_type=jnp.float32) * d16
    g2_ref[1] = jnp.dot(
        hcat, whi_ref[...], preferred_element_type=jnp.float32) * d16


def _mid_call(acc, g, d16, b1lo, b1hi, w2lo, w2hi):
    return pl.pallas_call(
        _mid_body,
        grid=(_G,),
        in_specs=[
            pl.BlockSpec((N_SC, _RP, 128), lambda i: (0, i, 0)),
            pl.BlockSpec((N_SC, _RP, 128), lambda i: (0, i, 0)),
            pl.BlockSpec((_RP, 128), lambda i: (i, 0)),
            pl.BlockSpec((1, 128), lambda i: (0, 0)),
            pl.BlockSpec((1, 128), lambda i: (0, 0)),
            pl.BlockSpec((256, 128), lambda i: (0, 0)),
            pl.BlockSpec((256, 128), lambda i: (0, 0)),
        ],
        out_specs=pl.BlockSpec((N_SC, _RP, 128), lambda i: (0, i, 0)),
        out_shape=jax.ShapeDtypeStruct((N_SC, _PK, 128), jnp.float32),
    )(acc, g, d16, b1lo, b1hi, w2lo, w2hi)


def _fin_body(acc_ref, g_ref, d16_ref, blo_ref, bhi_ref, wfc_ref, bfc_ref,
              out_ref):
    d16 = d16_ref[...]
    t_lo = jnp.maximum((acc_ref[0] + g_ref[0]) * d16 + blo_ref[...], 0.0)
    t_hi = jnp.maximum((acc_ref[1] + g_ref[1]) * d16 + bhi_ref[...], 0.0)
    hcat = jnp.concatenate([t_lo, t_hi], axis=1)
    out_ref[...] = jnp.dot(
        hcat, wfc_ref[...], preferred_element_type=jnp.float32) + bfc_ref[...]


def _fin_call(acc, g, d16, b2lo, b2hi, wfcb, bfcp):
    return pl.pallas_call(
        _fin_body,
        grid=(_G,),
        in_specs=[
            pl.BlockSpec((N_SC, _RP, 128), lambda i: (0, i, 0)),
            pl.BlockSpec((N_SC, _RP, 128), lambda i: (0, i, 0)),
            pl.BlockSpec((_RP, 128), lambda i: (i, 0)),
            pl.BlockSpec((1, 128), lambda i: (0, 0)),
            pl.BlockSpec((1, 128), lambda i: (0, 0)),
            pl.BlockSpec((256, 80), lambda i: (0, 0)),
            pl.BlockSpec((1, 80), lambda i: (0, 0)),
        ],
        out_specs=pl.BlockSpec((_RP, 80), lambda i: (i, 0)),
        out_shape=jax.ShapeDtypeStruct((_PK, 80), jnp.float32),
    )(acc, g, d16, b2lo, b2hi, wfcb, bfcp)


import numpy as np

_A_IN9 = np.asarray([c // 9 for c in range(72)])
_K_IN9 = np.asarray([c % 9 for c in range(72)])
_A_OUT = np.asarray([c // 16 for c in range(128)])
_F_OUT = np.asarray([c % 16 for c in range(128)])
_A_CAT = np.asarray([c // 16 for c in range(128)] * 2)
_K_CAT = np.asarray([c % 16 for c in range(128)]
                    + [16 + c % 16 for c in range(128)])
_A_FC = np.asarray([c // 10 for c in range(80)])
_F_FC = np.asarray([c % 10 for c in range(80)])


def _blkdiag(Wp, a_in, k_in, a_out, f_out):
    vals = Wp[k_in[:, None], f_out[None, :]]
    mask = jnp.asarray((a_in[:, None] == a_out[None, :]).astype(np.float32))
    return vals * mask


def _tile8(v, width):
    return jnp.tile(jnp.pad(v, (0, width - v.shape[0])), 8)[None, :]


@jax.jit
def kernel(x, edge_index, W1, b1, W2, b2, Wfc, bfc):
    ei = edge_index.astype(jnp.int32)
    src, dst = ei[0], ei[1]
    e = src.shape[0]
    ep = ((e + EDGE_ALIGN - 1) // EDGE_ALIGN) * EDGE_ALIGN
    pad = ep - e
    pad_i = jnp.arange(pad, dtype=jnp.int32)
    src_p = jnp.concatenate([src, (pad_i * 37) % N_NODES])
    dst_p = jnp.concatenate([dst, N_NODES + (pad_i % 16)])
    src2d = src_p.reshape(ep // 128, 128)
    dst2d = dst_p.reshape(ep // 128, 128)

    W1p = jnp.pad(W1, ((0, 0), (0, 5)))
    W2p = jnp.pad(W2, ((0, 5), (0, 5)))
    Wfcp = jnp.pad(Wfc, ((0, 5), (0, 0)))
    w1lo = _blkdiag(W1p, _A_IN9, _K_IN9, _A_OUT, _F_OUT)
    w1hi = _blkdiag(W1p, _A_IN9, _K_IN9, _A_OUT, _F_OUT + 16)
    w2lo = _blkdiag(W2p, _A_CAT, _K_CAT, _A_OUT, _F_OUT)
    w2hi = _blkdiag(W2p, _A_CAT, _K_CAT, _A_OUT, _F_OUT + 16)
    wfcb = _blkdiag(Wfcp, _A_CAT, _K_CAT, _A_FC, _F_FC)
    b1lo = _tile8(b1[:DH], DH)
    b1hi = _tile8(b1[DH:], DH)
    b2lo = _tile8(b2[:DH], DH)
    b2hi = _tile8(b2[DH:], DH)
    bfcp = _tile8(bfc, 10)
    xp = jnp.concatenate(
        [x, jnp.zeros((NPAD - N_NODES, 9), jnp.float32)]).reshape(_PK, 72)

    parts3 = _deg_call(dst_p).reshape(N_SC * N_TILE, _DK, 128)
    dinv128 = _dinv_call(parts3)
    d16 = jnp.repeat(dinv128.reshape(-1), DH).reshape(_PK, 128)
    g1 = _prep1_call(xp, d16, w1lo, w1hi)
    acc1 = _agg_call(g1.reshape(N_SC, NPAD, DH), src2d, dst2d)
    g2 = _mid_call(acc1.reshape(N_SC, _PK, 128), g1, d16,
                   b1lo, b1hi, w2lo, w2hi)
    acc2 = _agg_call(g2.reshape(N_SC, NPAD, DH), src2d, dst2d)
    outpk = _fin_call(acc2.reshape(N_SC, _PK, 128), g2, d16,
                      b2lo, b2hi, wfcb, bfcp)
    return outpk.reshape(NPAD, 10)[:N_NODES]


# speedup vs baseline: 1.1132x; 1.1132x over previous
import functools
import jax
import jax.numpy as jnp
from jax import lax
from jax.experimental import pallas as pl
from jax.experimental.pallas import tpu as pltpu
from jax.experimental.pallas import tpu_sc as plsc

N_NODES = 100000
NPAD = 100352
N_SC = 2
N_TILE = 16
DH = 16

EDGE_ALIGN = 32 * 2048

DEG_SLAB = 2048
AGG_CHUNK = 128
AGG_NCHUNK = 4
AGG_SLAB = AGG_CHUNK * AGG_NCHUNK
DEPTH = 4


def _sc_mesh():
    return plsc.VectorSubcoreMesh(
        core_axis_name="c", subcore_axis_name="s",
        num_cores=N_SC, num_subcores=N_TILE)


def _deg_body(dst_hbm, parts_hbm, dbuf, acc):
    ep = dst_hbm.shape[0]
    wid = lax.axis_index("s") * N_SC + lax.axis_index("c")
    ew = ep // (N_SC * N_TILE)
    base0 = wid * ew
    ones = jnp.full((16,), 1.0, jnp.float32)
    zeros = jnp.zeros((16,), jnp.float32)

    def zbody(i, _):
        acc[pl.ds(i * 16, 16)] = zeros
        return 0
    lax.fori_loop(0, NPAD // 16, zbody, 0)

    def slab(i, _):
        pltpu.sync_copy(dst_hbm.at[pl.ds(base0 + i * DEG_SLAB, DEG_SLAB)], dbuf)

        def inner(j, _):
            idx = dbuf[pl.ds(j * 16, 16)]
            plsc.addupdate_scatter(acc, [idx], ones)
            return 0
        lax.fori_loop(0, DEG_SLAB // 16, inner, 0)
        return 0
    lax.fori_loop(0, ew // DEG_SLAB, slab, 0)

    pltpu.sync_copy(acc, parts_hbm.at[pl.ds(wid * NPAD, NPAD)])


def _deg_call(dst_p):
    return pl.kernel(
        _deg_body,
        out_type=jax.ShapeDtypeStruct((N_SC * N_TILE * NPAD,), jnp.float32),
        mesh=_sc_mesh(),
        compiler_params=pltpu.CompilerParams(needs_layout_passes=False, use_tc_tiling_on_sc=False),
        scratch_types=[
            pltpu.VMEM((DEG_SLAB,), jnp.int32),
            pltpu.VMEM((NPAD,), jnp.float32),
        ],
    )(dst_p)


def _agg_body(g_hbm, src_hbm, dst_hbm, out_hbm,
              sbuf, dbuf, rows, acc_sh, semi0, semi1, semg, sems0, sems1):
    semi = (semi0, semi1)
    sems = (sems0, sems1)
    nrows_idx = src_hbm.shape[0]
    c = lax.axis_index("c")
    s = lax.axis_index("s")
    gtab = g_hbm.at[c]
    otab = out_hbm.at[c]

    zeros = jnp.zeros((16,), jnp.float32)

    def zfill(i, _):
        rows[0, i] = zeros
        return 0
    lax.fori_loop(0, AGG_SLAB, zfill, 0)

    stripe = NPAD // N_TILE
    r0 = s * stripe
    nz_full, nz_rem = divmod(stripe, AGG_SLAB)
    for k in range(nz_full):
        pltpu.sync_copy(rows.at[0],
                        acc_sh.at[pl.ds(r0 + k * AGG_SLAB, AGG_SLAB), :])
    if nz_rem:
        pltpu.sync_copy(rows.at[0].at[pl.ds(0, nz_rem), :],
                        acc_sh.at[pl.ds(r0 + nz_full * AGG_SLAB, nz_rem), :])
    plsc.subcore_barrier()

    rows_per_tile = nrows_idx // N_TILE
    nslab = rows_per_tile // AGG_NCHUNK
    irow0 = s * rows_per_tile

    def issue_idx(slot, i, p):
        ib = irow0 + i * AGG_NCHUNK
        pltpu.async_copy(src_hbm.at[pl.ds(ib, AGG_NCHUNK), :], sbuf.at[slot],
                         semi[p])
        pltpu.async_copy(dst_hbm.at[pl.ds(ib, AGG_NCHUNK), :], dbuf.at[slot],
                         semi[p])

    def wait_idx(slot, p):
        pltpu.make_async_copy(
            src_hbm.at[pl.ds(0, AGG_NCHUNK), :], sbuf.at[slot], semi[p]).wait()
        pltpu.make_async_copy(
            dst_hbm.at[pl.ds(0, AGG_NCHUNK), :], dbuf.at[slot], semi[p]).wait()

    def drain_scatter(slot, p):
        for k in range(AGG_NCHUNK):
            pltpu.make_async_copy(
                rows.at[p].at[pl.ds(k * AGG_CHUNK, AGG_CHUNK), :],
                acc_sh.at[dbuf.at[slot].at[k]], sems[p]).wait()

    issue_idx(0, 0, 0)
    issue_idx(1, 1, 1)

    def body(i4, _):
        for u in range(4):
            i = i4 * 4 + u
            p = u % 2

            @pl.when(i >= 2)
            def _():
                drain_scatter((u - 2) % 4, p)

            wait_idx(u, p)

            @pl.when(i + 2 < nslab)
            def _():
                issue_idx((u + 2) % 4, i + 2, p)

            gd = [
                pltpu.async_copy(
                    gtab.at[sbuf.at[u].at[k]],
                    rows.at[p].at[pl.ds(k * AGG_CHUNK, AGG_CHUNK), :], semg)
                for k in range(AGG_NCHUNK)
            ]
            for d in gd:
                d.wait()
            for k in range(AGG_NCHUNK):
                pltpu.async_copy(
                    rows.at[p].at[pl.ds(k * AGG_CHUNK, AGG_CHUNK), :],
                    acc_sh.at[dbuf.at[u].at[k]], sems[p], add=True)
        return 0
    lax.fori_loop(0, nslab // 4, body, 0)
    drain_scatter(2, 0)
    drain_scatter(3, 1)
    plsc.subcore_barrier()

    pltpu.sync_copy(acc_sh.at[pl.ds(r0, stripe), :], otab.at[pl.ds(r0, stripe), :])


def _agg_call(g, src2d, dst2d):
    return pl.kernel(
        _agg_body,
        out_type=jax.ShapeDtypeStruct((N_SC, NPAD, DH), jnp.float32),
        mesh=_sc_mesh(),
        compiler_params=pltpu.CompilerParams(needs_layout_passes=False, use_tc_tiling_on_sc=False),
        scratch_types=[
            pltpu.VMEM((DEPTH, AGG_NCHUNK, AGG_CHUNK), jnp.int32),
            pltpu.VMEM((DEPTH, AGG_NCHUNK, AGG_CHUNK), jnp.int32),
            pltpu.VMEM((2, AGG_SLAB, DH), jnp.float32),
            pltpu.VMEM_SHARED((NPAD, DH), jnp.float32),
            pltpu.SemaphoreType.DMA,
            pltpu.SemaphoreType.DMA,
            pltpu.SemaphoreType.DMA,
            pltpu.SemaphoreType.DMA,
            pltpu.SemaphoreType.DMA,
        ],
    )(g, src2d, dst2d)


_R = 2048
_G = (N_NODES + _R - 1) // _R


_PK = NPAD // 8
_RP = _R // 8
_DK = NPAD // 128
_GD = _DK // 16


def _dinv_body(parts_ref, dinv_ref):
    deg = jnp.sum(parts_ref[...], axis=0) + 1.0
    dinv_ref[...] = lax.rsqrt(deg)


def _dinv_call(parts3):
    return pl.pallas_call(
        _dinv_body,
        grid=(_GD,),
        in_specs=[pl.BlockSpec((N_SC * N_TILE, 16, 128), lambda i: (0, i, 0))],
        out_specs=pl.BlockSpec((16, 128), lambda i: (i, 0)),
        out_shape=jax.ShapeDtypeStruct((_DK, 128), jnp.float32),
    )(parts3)


def _prep1_body(xp_ref, d16_ref, wlo_ref, whi_ref, g_ref):
    xb = xp_ref[...]
    d16 = d16_ref[...]
    g_ref[0] = jnp.dot(xb, wlo_ref[...], preferred_element_type=jnp.float32) * d16
    g_ref[1] = jnp.dot(xb, whi_ref[...], preferred_element_type=jnp.float32) * d16


def _prep1_call(xp, d16, w1lo, w1hi):
    return pl.pallas_call(
        _prep1_body,
        grid=(_G,),
        in_specs=[
            pl.BlockSpec((_RP, 72), lambda i: (i, 0)),
            pl.BlockSpec((_RP, 128), lambda i: (i, 0)),
            pl.BlockSpec((72, 128), lambda i: (0, 0)),
            pl.BlockSpec((72, 128), lambda i: (0, 0)),
        ],
        out_specs=pl.BlockSpec((N_SC, _RP, 128), lambda i: (0, i, 0)),
        out_shape=jax.ShapeDtypeStruct((N_SC, _PK, 128), jnp.float32),
    )(xp, d16, w1lo, w1hi)


def _mid_body(acc_ref, g_ref, d16_ref, blo_ref, bhi_ref, wlo_ref, whi_ref,
              g2_ref):
    d16 = d16_ref[...]
    t_lo = jnp.maximum((acc_ref[0] + g_ref[0]) * d16 + blo_ref[...], 0.0)
    t_hi = jnp.maximum((acc_ref[1] + g_ref[1]) * d16 + bhi_ref[...], 0.0)
    hcat = jnp.concatenate([t_lo, t_hi], axis=1)
    g2_ref[0] = jnp.dot(
        hcat, wlo_ref[...], preferred_element_type=jnp.float32) * d16
    g2_ref[1] = jnp.dot(
        hcat, whi_ref[...], preferred_element_type=jnp.float32) * d16


def _mid_call(acc, g, d16, b1lo, b1hi, w2lo, w2hi):
    return pl.pallas_call(
        _mid_body,
        grid=(_G,),
        in_specs=[
            pl.BlockSpec((N_SC, _RP, 128), lambda i: (0, i, 0)),
            pl.BlockSpec((N_SC, _RP, 128), lambda i: (0, i, 0)),
            pl.BlockSpec((_RP, 128), lambda i: (i, 0)),
            pl.BlockSpec((1, 128), lambda i: (0, 0)),
            pl.BlockSpec((1, 128), lambda i: (0, 0)),
            pl.BlockSpec((256, 128), lambda i: (0, 0)),
            pl.BlockSpec((256, 128), lambda i: (0, 0)),
        ],
        out_specs=pl.BlockSpec((N_SC, _RP, 128), lambda i: (0, i, 0)),
        out_shape=jax.ShapeDtypeStruct((N_SC, _PK, 128), jnp.float32),
    )(acc, g, d16, b1lo, b1hi, w2lo, w2hi)


def _fin_body(acc_ref, g_ref, d16_ref, blo_ref, bhi_ref, wfc_ref, bfc_ref,
              out_ref):
    d16 = d16_ref[...]
    t_lo = jnp.maximum((acc_ref[0] + g_ref[0]) * d16 + blo_ref[...], 0.0)
    t_hi = jnp.maximum((acc_ref[1] + g_ref[1]) * d16 + bhi_ref[...], 0.0)
    hcat = jnp.concatenate([t_lo, t_hi], axis=1)
    out_ref[...] = jnp.dot(
        hcat, wfc_ref[...], preferred_element_type=jnp.float32) + bfc_ref[...]


def _fin_call(acc, g, d16, b2lo, b2hi, wfcb, bfcp):
    return pl.pallas_call(
        _fin_body,
        grid=(_G,),
        in_specs=[
            pl.BlockSpec((N_SC, _RP, 128), lambda i: (0, i, 0)),
            pl.BlockSpec((N_SC, _RP, 128), lambda i: (0, i, 0)),
            pl.BlockSpec((_RP, 128), lambda i: (i, 0)),
            pl.BlockSpec((1, 128), lambda i: (0, 0)),
            pl.BlockSpec((1, 128), lambda i: (0, 0)),
            pl.BlockSpec((256, 80), lambda i: (0, 0)),
            pl.BlockSpec((1, 80), lambda i: (0, 0)),
        ],
        out_specs=pl.BlockSpec((_RP, 80), lambda i: (i, 0)),
        out_shape=jax.ShapeDtypeStruct((_PK, 80), jnp.float32),
    )(acc, g, d16, b2lo, b2hi, wfcb, bfcp)


import numpy as np

_A_IN9 = np.asarray([c // 9 for c in range(72)])
_K_IN9 = np.asarray([c % 9 for c in range(72)])
_A_OUT = np.asarray([c // 16 for c in range(128)])
_F_OUT = np.asarray([c % 16 for c in range(128)])
_A_CAT = np.asarray([c // 16 for c in range(128)] * 2)
_K_CAT = np.asarray([c % 16 for c in range(128)]
                    + [16 + c % 16 for c in range(128)])
_A_FC = np.asarray([c // 10 for c in range(80)])
_F_FC = np.asarray([c % 10 for c in range(80)])


def _blkdiag(Wp, a_in, k_in, a_out, f_out):
    vals = Wp[k_in[:, None], f_out[None, :]]
    mask = jnp.asarray((a_in[:, None] == a_out[None, :]).astype(np.float32))
    return vals * mask


def _tile8(v, width):
    return jnp.tile(jnp.pad(v, (0, width - v.shape[0])), 8)[None, :]


@jax.jit
def kernel(x, edge_index, W1, b1, W2, b2, Wfc, bfc):
    ei = edge_index.astype(jnp.int32)
    src, dst = ei[0], ei[1]
    e = src.shape[0]
    ep = ((e + EDGE_ALIGN - 1) // EDGE_ALIGN) * EDGE_ALIGN
    pad = ep - e
    pad_i = jnp.arange(pad, dtype=jnp.int32)
    src_p = jnp.concatenate([src, (pad_i * 37) % N_NODES])
    dst_p = jnp.concatenate([dst, N_NODES + (pad_i % 16)])
    src2d = src_p.reshape(ep // 128, 128)
    dst2d = dst_p.reshape(ep // 128, 128)

    W1p = jnp.pad(W1, ((0, 0), (0, 5)))
    W2p = jnp.pad(W2, ((0, 5), (0, 5)))
    Wfcp = jnp.pad(Wfc, ((0, 5), (0, 0)))
    w1lo = _blkdiag(W1p, _A_IN9, _K_IN9, _A_OUT, _F_OUT)
    w1hi = _blkdiag(W1p, _A_IN9, _K_IN9, _A_OUT, _F_OUT + 16)
    w2lo = _blkdiag(W2p, _A_CAT, _K_CAT, _A_OUT, _F_OUT)
    w2hi = _blkdiag(W2p, _A_CAT, _K_CAT, _A_OUT, _F_OUT + 16)
    wfcb = _blkdiag(Wfcp, _A_CAT, _K_CAT, _A_FC, _F_FC)
    b1lo = _tile8(b1[:DH], DH)
    b1hi = _tile8(b1[DH:], DH)
    b2lo = _tile8(b2[:DH], DH)
    b2hi = _tile8(b2[DH:], DH)
    bfcp = _tile8(bfc, 10)
    xp = jnp.concatenate(
        [x, jnp.zeros((NPAD - N_NODES, 9), jnp.float32)]).reshape(_PK, 72)

    parts3 = _deg_call(dst_p).reshape(N_SC * N_TILE, _DK, 128)
    dinv128 = _dinv_call(parts3)
    d16 = jnp.repeat(dinv128.reshape(-1), DH).reshape(_PK, 128)
    g1 = _prep1_call(xp, d16, w1lo, w1hi)
    acc1 = _agg_call(g1.reshape(N_SC, NPAD, DH), src2d, dst2d)
    g2 = _mid_call(acc1.reshape(N_SC, _PK, 128), g1, d16,
                   b1lo, b1hi, w2lo, w2hi)
    acc2 = _agg_call(g2.reshape(N_SC, NPAD, DH), src2d, dst2d)
    outpk = _fin_call(acc2.reshape(N_SC, _PK, 128), g2, d16,
                      b2lo, b2hi, wfcb, bfcp)
    return outpk.reshape(NPAD, 10)[:N_NODES]


# speedup vs baseline: 1.1135x; 1.0003x over previous
import functools
import jax
import jax.numpy as jnp
from jax import lax
from jax.experimental import pallas as pl
from jax.experimental.pallas import tpu as pltpu
from jax.experimental.pallas import tpu_sc as plsc

N_NODES = 100000
NPAD = 100352
N_SC = 2
N_TILE = 16
DH = 16

EDGE_ALIGN = 32 * 2048

DEG_SLAB = 2048
AGG_CHUNK = 128
AGG_NCHUNK = 4
AGG_SLAB = AGG_CHUNK * AGG_NCHUNK
DEPTH = 4


def _sc_mesh():
    return plsc.VectorSubcoreMesh(
        core_axis_name="c", subcore_axis_name="s",
        num_cores=N_SC, num_subcores=N_TILE)


def _deg_body(dst_hbm, parts_hbm, dbuf, acc):
    ep = dst_hbm.shape[0]
    wid = lax.axis_index("s") * N_SC + lax.axis_index("c")
    ew = ep // (N_SC * N_TILE)
    base0 = wid * ew
    ones = jnp.full((16,), 1.0, jnp.float32)
    zeros = jnp.zeros((16,), jnp.float32)

    def zbody(i, _):
        acc[pl.ds(i * 16, 16)] = zeros
        return 0
    lax.fori_loop(0, NPAD // 16, zbody, 0)

    def slab(i, _):
        pltpu.sync_copy(dst_hbm.at[pl.ds(base0 + i * DEG_SLAB, DEG_SLAB)], dbuf)

        def inner(j, _):
            for t in range(4):
                idx = dbuf[pl.ds(j * 64 + t * 16, 16)]
                plsc.addupdate_scatter(acc, [idx], ones)
            return 0
        lax.fori_loop(0, DEG_SLAB // 64, inner, 0)
        return 0
    lax.fori_loop(0, ew // DEG_SLAB, slab, 0)

    pltpu.sync_copy(acc, parts_hbm.at[pl.ds(wid * NPAD, NPAD)])


def _deg_call(dst_p):
    return pl.kernel(
        _deg_body,
        out_type=jax.ShapeDtypeStruct((N_SC * N_TILE * NPAD,), jnp.float32),
        mesh=_sc_mesh(),
        compiler_params=pltpu.CompilerParams(needs_layout_passes=False, use_tc_tiling_on_sc=False),
        scratch_types=[
            pltpu.VMEM((DEG_SLAB,), jnp.int32),
            pltpu.VMEM((NPAD,), jnp.float32),
        ],
    )(dst_p)


def _agg_body(g_hbm, src_hbm, dst_hbm, out_hbm,
              sbuf, dbuf, rows, acc_sh, semi0, semi1, semg, sems0, sems1):
    semi = (semi0, semi1)
    sems = (sems0, sems1)
    nrows_idx = src_hbm.shape[0]
    c = lax.axis_index("c")
    s = lax.axis_index("s")
    gtab = g_hbm.at[c]
    otab = out_hbm.at[c]

    zeros = jnp.zeros((16,), jnp.float32)

    def zfill(i, _):
        rows[0, i] = zeros
        return 0
    lax.fori_loop(0, AGG_SLAB, zfill, 0)

    stripe = NPAD // N_TILE
    r0 = s * stripe
    nz_full, nz_rem = divmod(stripe, AGG_SLAB)
    for k in range(nz_full):
        pltpu.sync_copy(rows.at[0],
                        acc_sh.at[pl.ds(r0 + k * AGG_SLAB, AGG_SLAB), :])
    if nz_rem:
        pltpu.sync_copy(rows.at[0].at[pl.ds(0, nz_rem), :],
                        acc_sh.at[pl.ds(r0 + nz_full * AGG_SLAB, nz_rem), :])
    plsc.subcore_barrier()

    rows_per_tile = nrows_idx // N_TILE
    nslab = rows_per_tile // AGG_NCHUNK
    irow0 = s * rows_per_tile

    def issue_idx(slot, i, p):
        ib = irow0 + i * AGG_NCHUNK
        pltpu.async_copy(src_hbm.at[pl.ds(ib, AGG_NCHUNK), :], sbuf.at[slot],
                         semi[p])
        pltpu.async_copy(dst_hbm.at[pl.ds(ib, AGG_NCHUNK), :], dbuf.at[slot],
                         semi[p])

    def wait_idx(slot, p):
        pltpu.make_async_copy(
            src_hbm.at[pl.ds(0, AGG_NCHUNK), :], sbuf.at[slot], semi[p]).wait()
        pltpu.make_async_copy(
            dst_hbm.at[pl.ds(0, AGG_NCHUNK), :], dbuf.at[slot], semi[p]).wait()

    def drain_scatter(slot, p):
        for k in range(AGG_NCHUNK):
            pltpu.make_async_copy(
                rows.at[p].at[pl.ds(k * AGG_CHUNK, AGG_CHUNK), :],
                acc_sh.at[dbuf.at[slot].at[k]], sems[p]).wait()

    issue_idx(0, 0, 0)
    issue_idx(1, 1, 1)

    def body(i4, _):
        for u in range(4):
            i = i4 * 4 + u
            p = u % 2

            @pl.when(i >= 2)
            def _():
                drain_scatter((u - 2) % 4, p)

            wait_idx(u, p)

            @pl.when(i + 2 < nslab)
            def _():
                issue_idx((u + 2) % 4, i + 2, p)

            gd = [
                pltpu.async_copy(
                    gtab.at[sbuf.at[u].at[k]],
                    rows.at[p].at[pl.ds(k * AGG_CHUNK, AGG_CHUNK), :], semg)
                for k in range(AGG_NCHUNK)
            ]
            for d in gd:
                d.wait()
            for k in range(AGG_NCHUNK):
                pltpu.async_copy(
                    rows.at[p].at[pl.ds(k * AGG_CHUNK, AGG_CHUNK), :],
                    acc_sh.at[dbuf.at[u].at[k]], sems[p], add=True)
        return 0
    lax.fori_loop(0, nslab // 4, body, 0)
    drain_scatter(2, 0)
    drain_scatter(3, 1)
    plsc.subcore_barrier()

    pltpu.sync_copy(acc_sh.at[pl.ds(r0, stripe), :], otab.at[pl.ds(r0, stripe), :])


def _agg_call(g, src2d, dst2d):
    return pl.kernel(
        _agg_body,
        out_type=jax.ShapeDtypeStruct((N_SC, NPAD, DH), jnp.float32),
        mesh=_sc_mesh(),
        compiler_params=pltpu.CompilerParams(needs_layout_passes=False, use_tc_tiling_on_sc=False),
        scratch_types=[
            pltpu.VMEM((DEPTH, AGG_NCHUNK, AGG_CHUNK), jnp.int32),
            pltpu.VMEM((DEPTH, AGG_NCHUNK, AGG_CHUNK), jnp.int32),
            pltpu.VMEM((2, AGG_SLAB, DH), jnp.float32),
            pltpu.VMEM_SHARED((NPAD, DH), jnp.float32),
            pltpu.SemaphoreType.DMA,
            pltpu.SemaphoreType.DMA,
            pltpu.SemaphoreType.DMA,
            pltpu.SemaphoreType.DMA,
            pltpu.SemaphoreType.DMA,
        ],
    )(g, src2d, dst2d)


_R = 2048
_G = (N_NODES + _R - 1) // _R


_PK = NPAD // 8
_RP = _R // 8
_DK = NPAD // 128
_GD = _DK // 16


def _dinv_body(parts_ref, dinv_ref):
    deg = jnp.sum(parts_ref[...], axis=0) + 1.0
    dinv_ref[...] = lax.rsqrt(deg)


def _dinv_call(parts3):
    return pl.pallas_call(
        _dinv_body,
        grid=(_GD,),
        in_specs=[pl.BlockSpec((N_SC * N_TILE, 16, 128), lambda i: (0, i, 0))],
        out_specs=pl.BlockSpec((16, 128), lambda i: (i, 0)),
        out_shape=jax.ShapeDtypeStruct((_DK, 128), jnp.float32),
    )(parts3)


def _prep1_body(xp_ref, d16_ref, wlo_ref, whi_ref, g_ref):
    xb = xp_ref[...]
    d16 = d16_ref[...]
    g_ref[0] = jnp.dot(xb, wlo_ref[...], preferred_element_type=jnp.float32) * d16
    g_ref[1] = jnp.dot(xb, whi_ref[...], preferred_element_type=jnp.float32) * d16


def _prep1_call(xp, d16, w1lo, w1hi):
    return pl.pallas_call(
        _prep1_body,
        grid=(_G,),
        in_specs=[
            pl.BlockSpec((_RP, 72), lambda i: (i, 0)),
            pl.BlockSpec((_RP, 128), lambda i: (i, 0)),
            pl.BlockSpec((72, 128), lambda i: (0, 0)),
            pl.BlockSpec((72, 128), lambda i: (0, 0)),
        ],
        out_specs=pl.BlockSpec((N_SC, _RP, 128), lambda i: (0, i, 0)),
        out_shape=jax.ShapeDtypeStruct((N_SC, _PK, 128), jnp.float32),
    )(xp, d16, w1lo, w1hi)


def _mid_body(acc_ref, g_ref, d16_ref, blo_ref, bhi_ref, wlo_ref, whi_ref,
              g2_ref):
    d16 = d16_ref[...]
    t_lo = jnp.maximum((acc_ref[0] + g_ref[0]) * d16 + blo_ref[...], 0.0)
    t_hi = jnp.maximum((acc_ref[1] + g_ref[1]) * d16 + bhi_ref[...], 0.0)
    hcat = jnp.concatenate([t_lo, t_hi], axis=1)
    g2_ref[0] = jnp.dot(
        hcat, wlo_ref[...], preferred_element_type=jnp.float32) * d16
    g2_ref[1] = jnp.dot(
        hcat, whi_ref[...], preferred_element_type=jnp.float32) * d16


def _mid_call(acc, g, d16, b1lo, b1hi, w2lo, w2hi):
    return pl.pallas_call(
        _mid_body,
        grid=(_G,),
        in_specs=[
            pl.BlockSpec((N_SC, _RP, 128), lambda i: (0, i, 0)),
            pl.BlockSpec((N_SC, _RP, 128), lambda i: (0, i, 0)),
            pl.BlockSpec((_RP, 128), lambda i: (i, 0)),
            pl.BlockSpec((1, 128), lambda i: (0, 0)),
            pl.BlockSpec((1, 128), lambda i: (0, 0)),
            pl.BlockSpec((256, 128), lambda i: (0, 0)),
            pl.BlockSpec((256, 128), lambda i: (0, 0)),
        ],
        out_specs=pl.BlockSpec((N_SC, _RP, 128), lambda i: (0, i, 0)),
        out_shape=jax.ShapeDtypeStruct((N_SC, _PK, 128), jnp.float32),
    )(acc, g, d16, b1lo, b1hi, w2lo, w2hi)


def _fin_body(acc_ref, g_ref, d16_ref, blo_ref, bhi_ref, wfc_ref, bfc_ref,
              out_ref):
    d16 = d16_ref[...]
    t_lo = jnp.maximum((acc_ref[0] + g_ref[0]) * d16 + blo_ref[...], 0.0)
    t_hi = jnp.maximum((acc_ref[1] + g_ref[1]) * d16 + bhi_ref[...], 0.0)
    hcat = jnp.concatenate([t_lo, t_hi], axis=1)
    out_ref[...] = jnp.dot(
        hcat, wfc_ref[...], preferred_element_type=jnp.float32) + bfc_ref[...]


def _fin_call(acc, g, d16, b2lo, b2hi, wfcb, bfcp):
    return pl.pallas_call(
        _fin_body,
        grid=(_G,),
        in_specs=[
            pl.BlockSpec((N_SC, _RP, 128), lambda i: (0, i, 0)),
            pl.BlockSpec((N_SC, _RP, 128), lambda i: (0, i, 0)),
            pl.BlockSpec((_RP, 128), lambda i: (i, 0)),
            pl.BlockSpec((1, 128), lambda i: (0, 0)),
            pl.BlockSpec((1, 128), lambda i: (0, 0)),
            pl.BlockSpec((256, 80), lambda i: (0, 0)),
            pl.BlockSpec((1, 80), lambda i: (0, 0)),
        ],
        out_specs=pl.BlockSpec((_RP, 80), lambda i: (i, 0)),
        out_shape=jax.ShapeDtypeStruct((_PK, 80), jnp.float32),
    )(acc, g, d16, b2lo, b2hi, wfcb, bfcp)


import numpy as np

_A_IN9 = np.asarray([c // 9 for c in range(72)])
_K_IN9 = np.asarray([c % 9 for c in range(72)])
_A_OUT = np.asarray([c // 16 for c in range(128)])
_F_OUT = np.asarray([c % 16 for c in range(128)])
_A_CAT = np.asarray([c // 16 for c in range(128)] * 2)
_K_CAT = np.asarray([c % 16 for c in range(128)]
                    + [16 + c % 16 for c in range(128)])
_A_FC = np.asarray([c // 10 for c in range(80)])
_F_FC = np.asarray([c % 10 for c in range(80)])


def _blkdiag(Wp, a_in, k_in, a_out, f_out):
    vals = Wp[k_in[:, None], f_out[None, :]]
    mask = jnp.asarray((a_in[:, None] == a_out[None, :]).astype(np.float32))
    return vals * mask


def _tile8(v, width):
    return jnp.tile(jnp.pad(v, (0, width - v.shape[0])), 8)[None, :]


@jax.jit
def kernel(x, edge_index, W1, b1, W2, b2, Wfc, bfc):
    ei = edge_index.astype(jnp.int32)
    src, dst = ei[0], ei[1]
    e = src.shape[0]
    ep = ((e + EDGE_ALIGN - 1) // EDGE_ALIGN) * EDGE_ALIGN
    pad = ep - e
    pad_i = jnp.arange(pad, dtype=jnp.int32)
    src_p = jnp.concatenate([src, (pad_i * 37) % N_NODES])
    dst_p = jnp.concatenate([dst, N_NODES + (pad_i % 16)])
    src2d = src_p.reshape(ep // 128, 128)
    dst2d = dst_p.reshape(ep // 128, 128)

    W1p = jnp.pad(W1, ((0, 0), (0, 5)))
    W2p = jnp.pad(W2, ((0, 5), (0, 5)))
    Wfcp = jnp.pad(Wfc, ((0, 5), (0, 0)))
    w1lo = _blkdiag(W1p, _A_IN9, _K_IN9, _A_OUT, _F_OUT)
    w1hi = _blkdiag(W1p, _A_IN9, _K_IN9, _A_OUT, _F_OUT + 16)
    w2lo = _blkdiag(W2p, _A_CAT, _K_CAT, _A_OUT, _F_OUT)
    w2hi = _blkdiag(W2p, _A_CAT, _K_CAT, _A_OUT, _F_OUT + 16)
    wfcb = _blkdiag(Wfcp, _A_CAT, _K_CAT, _A_FC, _F_FC)
    b1lo = _tile8(b1[:DH], DH)
    b1hi = _tile8(b1[DH:], DH)
    b2lo = _tile8(b2[:DH], DH)
    b2hi = _tile8(b2[DH:], DH)
    bfcp = _tile8(bfc, 10)
    xp = jnp.concatenate(
        [x, jnp.zeros((NPAD - N_NODES, 9), jnp.float32)]).reshape(_PK, 72)

    parts3 = _deg_call(dst_p).reshape(N_SC * N_TILE, _DK, 128)
    dinv128 = _dinv_call(parts3)
    d16 = jnp.repeat(dinv128.reshape(-1), DH).reshape(_PK, 128)
    g1 = _prep1_call(xp, d16, w1lo, w1hi)
    acc1 = _agg_call(g1.reshape(N_SC, NPAD, DH), src2d, dst2d)
    g2 = _mid_call(acc1.reshape(N_SC, _PK, 128), g1, d16,
                   b1lo, b1hi, w2lo, w2hi)
    acc2 = _agg_call(g2.reshape(N_SC, NPAD, DH), src2d, dst2d)
    outpk = _fin_call(acc2.reshape(N_SC, _PK, 128), g2, d16,
                      b2lo, b2hi, wfcb, bfcp)
    return outpk.reshape(NPAD, 10)[:N_NODES]


# speedup vs baseline: 1.1149x; 1.0013x over previous
import jax
import jax.numpy as jnp
from jax import lax
from jax.experimental import pallas as pl
from jax.experimental.pallas import tpu as pltpu
from jax.experimental.pallas import tpu_sc as plsc

N_NODES = 100000
NPAD = 100352
N_SC = 2
N_TILE = 16
DH = 16

EDGE_ALIGN = 32 * 2048

DEG_SLAB = 2048
AGG_CHUNK = 128
AGG_NCHUNK = 4
AGG_SLAB = AGG_CHUNK * AGG_NCHUNK
DEPTH = 4


def _sc_mesh():
    return plsc.VectorSubcoreMesh(
        core_axis_name="c", subcore_axis_name="s",
        num_cores=N_SC, num_subcores=N_TILE)


def _deg_body(dst_hbm, parts_hbm, dbuf, acc):
    ep = dst_hbm.shape[0]
    wid = lax.axis_index("s") * N_SC + lax.axis_index("c")
    ew = ep // (N_SC * N_TILE)
    base0 = wid * ew
    ones = jnp.full((16,), 1.0, jnp.float32)
    zeros = jnp.zeros((16,), jnp.float32)

    def zbody(i, _):
        acc[pl.ds(i * 16, 16)] = zeros
        return 0
    lax.fori_loop(0, NPAD // 16, zbody, 0)

    def slab(i, _):
        pltpu.sync_copy(dst_hbm.at[pl.ds(base0 + i * DEG_SLAB, DEG_SLAB)], dbuf)

        def inner(j, _):
            for t in range(4):
                idx = dbuf[pl.ds(j * 64 + t * 16, 16)]
                plsc.addupdate_scatter(acc, [idx], ones)
            return 0
        lax.fori_loop(0, DEG_SLAB // 64, inner, 0)
        return 0
    lax.fori_loop(0, ew // DEG_SLAB, slab, 0)

    pltpu.sync_copy(acc, parts_hbm.at[pl.ds(wid * NPAD, NPAD)])


def _deg_call(dst_p):
    return pl.kernel(
        _deg_body,
        out_type=jax.ShapeDtypeStruct((N_SC * N_TILE * NPAD,), jnp.float32),
        mesh=_sc_mesh(),
        compiler_params=pltpu.CompilerParams(needs_layout_passes=False, use_tc_tiling_on_sc=False),
        scratch_types=[
            pltpu.VMEM((DEG_SLAB,), jnp.int32),
            pltpu.VMEM((NPAD,), jnp.float32),
        ],
    )(dst_p)


def _agg_body(g_hbm, src_hbm, dst_hbm, out_hbm,
              sbuf, dbuf, rows, acc_sh, semi0, semi1, semg, sems0, sems1):
    semi = (semi0, semi1)
    sems = (sems0, sems1)
    nrows_idx = src_hbm.shape[0]
    c = lax.axis_index("c")
    s = lax.axis_index("s")
    gtab = g_hbm.at[c]
    otab = out_hbm.at[c]

    zeros = jnp.zeros((16,), jnp.float32)

    def zfill(i, _):
        rows[0, i] = zeros
        return 0
    lax.fori_loop(0, AGG_SLAB, zfill, 0)

    stripe = NPAD // N_TILE
    r0 = s * stripe
    nz_full, nz_rem = divmod(stripe, AGG_SLAB)
    for k in range(nz_full):
        pltpu.sync_copy(rows.at[0],
                        acc_sh.at[pl.ds(r0 + k * AGG_SLAB, AGG_SLAB), :])
    if nz_rem:
        pltpu.sync_copy(rows.at[0].at[pl.ds(0, nz_rem), :],
                        acc_sh.at[pl.ds(r0 + nz_full * AGG_SLAB, nz_rem), :])
    plsc.subcore_barrier()

    rows_per_tile = nrows_idx // N_TILE
    nslab = rows_per_tile // AGG_NCHUNK
    irow0 = s * rows_per_tile

    def issue_idx(slot, i, p):
        ib = irow0 + i * AGG_NCHUNK
        pltpu.async_copy(src_hbm.at[pl.ds(ib, AGG_NCHUNK), :], sbuf.at[slot],
                         semi[p])
        pltpu.async_copy(dst_hbm.at[pl.ds(ib, AGG_NCHUNK), :], dbuf.at[slot],
                         semi[p])

    def wait_idx(slot, p):
        pltpu.make_async_copy(
            src_hbm.at[pl.ds(0, AGG_NCHUNK), :], sbuf.at[slot], semi[p]).wait()
        pltpu.make_async_copy(
            dst_hbm.at[pl.ds(0, AGG_NCHUNK), :], dbuf.at[slot], semi[p]).wait()

    def drain_scatter(slot, p):
        for k in range(AGG_NCHUNK):
            pltpu.make_async_copy(
                rows.at[p].at[pl.ds(k * AGG_CHUNK, AGG_CHUNK), :],
                acc_sh.at[dbuf.at[slot].at[k]], sems[p]).wait()

    issue_idx(0, 0, 0)
    issue_idx(1, 1, 1)

    def body(i4, _):
        for u in range(4):
            i = i4 * 4 + u
            p = u % 2

            @pl.when(i >= 2)
            def _():
                drain_scatter((u - 2) % 4, p)

            wait_idx(u, p)

            @pl.when(i + 2 < nslab)
            def _():
                issue_idx((u + 2) % 4, i + 2, p)

            gd = [
                pltpu.async_copy(
                    gtab.at[sbuf.at[u].at[k]],
                    rows.at[p].at[pl.ds(k * AGG_CHUNK, AGG_CHUNK), :], semg)
                for k in range(AGG_NCHUNK)
            ]
            for d in gd:
                d.wait()
            for k in range(AGG_NCHUNK):
                pltpu.async_copy(
                    rows.at[p].at[pl.ds(k * AGG_CHUNK, AGG_CHUNK), :],
                    acc_sh.at[dbuf.at[u].at[k]], sems[p], add=True)
        return 0
    lax.fori_loop(0, nslab // 4, body, 0)
    drain_scatter(2, 0)
    drain_scatter(3, 1)
    plsc.subcore_barrier()

    pltpu.sync_copy(acc_sh.at[pl.ds(r0, stripe), :], otab.at[pl.ds(r0, stripe), :])


def _agg_call(g, src2d, dst2d):
    return pl.kernel(
        _agg_body,
        out_type=jax.ShapeDtypeStruct((N_SC, NPAD, DH), jnp.float32),
        mesh=_sc_mesh(),
        compiler_params=pltpu.CompilerParams(needs_layout_passes=False, use_tc_tiling_on_sc=False),
        scratch_types=[
            pltpu.VMEM((DEPTH, AGG_NCHUNK, AGG_CHUNK), jnp.int32),
            pltpu.VMEM((DEPTH, AGG_NCHUNK, AGG_CHUNK), jnp.int32),
            pltpu.VMEM((2, AGG_SLAB, DH), jnp.float32),
            pltpu.VMEM_SHARED((NPAD, DH), jnp.float32),
            pltpu.SemaphoreType.DMA,
            pltpu.SemaphoreType.DMA,
            pltpu.SemaphoreType.DMA,
            pltpu.SemaphoreType.DMA,
            pltpu.SemaphoreType.DMA,
        ],
    )(g, src2d, dst2d)


_R = 2048
_G = (N_NODES + _R - 1) // _R


_PK = NPAD // 8
_RP = _R // 8
_DK = NPAD // 128
_GD = _DK // 16


def _dinv_body(parts_ref, dinv_ref):
    deg = jnp.sum(parts_ref[...], axis=0) + 1.0
    dinv_ref[...] = lax.rsqrt(deg)


def _dinv_call(parts3):
    return pl.pallas_call(
        _dinv_body,
        grid=(_GD,),
        in_specs=[pl.BlockSpec((N_SC * N_TILE, 16, 128), lambda i: (0, i, 0))],
        out_specs=pl.BlockSpec((16, 128), lambda i: (i, 0)),
        out_shape=jax.ShapeDtypeStruct((_DK, 128), jnp.float32),
    )(parts3)


def _prep1_body(xp_ref, d16_ref, wlo_ref, whi_ref, g_ref):
    xb = xp_ref[...]
    d16 = d16_ref[...]
    g_ref[0] = jnp.dot(xb, wlo_ref[...], preferred_element_type=jnp.float32) * d16
    g_ref[1] = jnp.dot(xb, whi_ref[...], preferred_element_type=jnp.float32) * d16


def _prep1_call(xp, d16, w1lo, w1hi):
    return pl.pallas_call(
        _prep1_body,
        grid=(_G,),
        in_specs=[
            pl.BlockSpec((_RP, 72), lambda i: (i, 0)),
            pl.BlockSpec((_RP, 128), lambda i: (i, 0)),
            pl.BlockSpec((72, 128), lambda i: (0, 0)),
            pl.BlockSpec((72, 128), lambda i: (0, 0)),
        ],
        out_specs=pl.BlockSpec((N_SC, _RP, 128), lambda i: (0, i, 0)),
        out_shape=jax.ShapeDtypeStruct((N_SC, _PK, 128), jnp.float32),
    )(xp, d16, w1lo, w1hi)


def _mid_body(acc_ref, g_ref, d16_ref, blo_ref, bhi_ref, wlo_ref, whi_ref,
              g2_ref):
    d16 = d16_ref[...]
    t_lo = jnp.maximum((acc_ref[0] + g_ref[0]) * d16 + blo_ref[...], 0.0)
    t_hi = jnp.maximum((acc_ref[1] + g_ref[1]) * d16 + bhi_ref[...], 0.0)
    hcat = jnp.concatenate([t_lo, t_hi], axis=1)
    g2_ref[0] = jnp.dot(
        hcat, wlo_ref[...], preferred_element_type=jnp.float32) * d16
    g2_ref[1] = jnp.dot(
        hcat, whi_ref[...], preferred_element_type=jnp.float32) * d16


def _mid_call(acc, g, d16, b1lo, b1hi, w2lo, w2hi):
    return pl.pallas_call(
        _mid_body,
        grid=(_G,),
        in_specs=[
            pl.BlockSpec((N_SC, _RP, 128), lambda i: (0, i, 0)),
            pl.BlockSpec((N_SC, _RP, 128), lambda i: (0, i, 0)),
            pl.BlockSpec((_RP, 128), lambda i: (i, 0)),
            pl.BlockSpec((1, 128), lambda i: (0, 0)),
            pl.BlockSpec((1, 128), lambda i: (0, 0)),
            pl.BlockSpec((256, 128), lambda i: (0, 0)),
            pl.BlockSpec((256, 128), lambda i: (0, 0)),
        ],
        out_specs=pl.BlockSpec((N_SC, _RP, 128), lambda i: (0, i, 0)),
        out_shape=jax.ShapeDtypeStruct((N_SC, _PK, 128), jnp.float32),
    )(acc, g, d16, b1lo, b1hi, w2lo, w2hi)


def _fin_body(acc_ref, g_ref, d16_ref, blo_ref, bhi_ref, wfc_ref, bfc_ref,
              out_ref):
    d16 = d16_ref[...]
    t_lo = jnp.maximum((acc_ref[0] + g_ref[0]) * d16 + blo_ref[...], 0.0)
    t_hi = jnp.maximum((acc_ref[1] + g_ref[1]) * d16 + bhi_ref[...], 0.0)
    hcat = jnp.concatenate([t_lo, t_hi], axis=1)
    out_ref[...] = jnp.dot(
        hcat, wfc_ref[...], preferred_element_type=jnp.float32) + bfc_ref[...]


def _fin_call(acc, g, d16, b2lo, b2hi, wfcb, bfcp):
    return pl.pallas_call(
        _fin_body,
        grid=(_G,),
        in_specs=[
            pl.BlockSpec((N_SC, _RP, 128), lambda i: (0, i, 0)),
            pl.BlockSpec((N_SC, _RP, 128), lambda i: (0, i, 0)),
            pl.BlockSpec((_RP, 128), lambda i: (i, 0)),
            pl.BlockSpec((1, 128), lambda i: (0, 0)),
            pl.BlockSpec((1, 128), lambda i: (0, 0)),
            pl.BlockSpec((256, 80), lambda i: (0, 0)),
            pl.BlockSpec((1, 80), lambda i: (0, 0)),
        ],
        out_specs=pl.BlockSpec((_RP, 80), lambda i: (i, 0)),
        out_shape=jax.ShapeDtypeStruct((_PK, 80), jnp.float32),
    )(acc, g, d16, b2lo, b2hi, wfcb, bfcp)


import numpy as np

_A_IN9 = np.asarray([c // 9 for c in range(72)])
_K_IN9 = np.asarray([c % 9 for c in range(72)])
_A_OUT = np.asarray([c // 16 for c in range(128)])
_F_OUT = np.asarray([c % 16 for c in range(128)])
_A_CAT = np.asarray([c // 16 for c in range(128)] * 2)
_K_CAT = np.asarray([c % 16 for c in range(128)]
                    + [16 + c % 16 for c in range(128)])
_A_FC = np.asarray([c // 10 for c in range(80)])
_F_FC = np.asarray([c % 10 for c in range(80)])


def _blkdiag(Wp, a_in, k_in, a_out, f_out):
    vals = Wp[k_in[:, None], f_out[None, :]]
    mask = jnp.asarray((a_in[:, None] == a_out[None, :]).astype(np.float32))
    return vals * mask


def _tile8(v, width):
    return jnp.tile(jnp.pad(v, (0, width - v.shape[0])), 8)[None, :]


@jax.jit
def kernel(x, edge_index, W1, b1, W2, b2, Wfc, bfc):
    ei = edge_index.astype(jnp.int32)
    src, dst = ei[0], ei[1]
    e = src.shape[0]
    ep = ((e + EDGE_ALIGN - 1) // EDGE_ALIGN) * EDGE_ALIGN
    pad = ep - e
    pad_i = jnp.arange(pad, dtype=jnp.int32)
    src_p = jnp.concatenate([src, (pad_i * 37) % N_NODES])
    dst_p = jnp.concatenate([dst, N_NODES + (pad_i % 16)])
    src2d = src_p.reshape(ep // 128, 128)
    dst2d = dst_p.reshape(ep // 128, 128)

    W1p = jnp.pad(W1, ((0, 0), (0, 5)))
    W2p = jnp.pad(W2, ((0, 5), (0, 5)))
    Wfcp = jnp.pad(Wfc, ((0, 5), (0, 0)))
    w1lo = _blkdiag(W1p, _A_IN9, _K_IN9, _A_OUT, _F_OUT)
    w1hi = _blkdiag(W1p, _A_IN9, _K_IN9, _A_OUT, _F_OUT + 16)
    w2lo = _blkdiag(W2p, _A_CAT, _K_CAT, _A_OUT, _F_OUT)
    w2hi = _blkdiag(W2p, _A_CAT, _K_CAT, _A_OUT, _F_OUT + 16)
    wfcb = _blkdiag(Wfcp, _A_CAT, _K_CAT, _A_FC, _F_FC)
    b1lo = _tile8(b1[:DH], DH)
    b1hi = _tile8(b1[DH:], DH)
    b2lo = _tile8(b2[:DH], DH)
    b2hi = _tile8(b2[DH:], DH)
    bfcp = _tile8(bfc, 10)
    xp = jnp.concatenate(
        [x, jnp.zeros((NPAD - N_NODES, 9), jnp.float32)]).reshape(_PK, 72)

    parts3 = _deg_call(dst_p).reshape(N_SC * N_TILE, _DK, 128)
    dinv128 = _dinv_call(parts3)
    d16 = jnp.repeat(dinv128.reshape(-1), DH).reshape(_PK, 128)
    g1 = _prep1_call(xp, d16, w1lo, w1hi)
    acc1 = _agg_call(g1.reshape(N_SC, NPAD, DH), src2d, dst2d)
    g2 = _mid_call(acc1.reshape(N_SC, _PK, 128), g1, d16,
                   b1lo, b1hi, w2lo, w2hi)
    acc2 = _agg_call(g2.reshape(N_SC, NPAD, DH), src2d, dst2d)
    outpk = _fin_call(acc2.reshape(N_SC, _PK, 128), g2, d16,
                      b2lo, b2hi, wfcb, bfcp)
    return outpk.reshape(NPAD, 10)[:N_NODES]
